# baseline (speedup 1.0000x reference)
import functools

import jax
import jax.numpy as jnp
from jax.experimental import pallas as pl
from jax.experimental.pallas import tpu as pltpu
from jax.experimental.pallas import tpu_sc as plsc

_GATHER_WIN = 128
_VPAD = 100096
_TABLE_BLK = 4352


def _eye(k):
    r = jax.lax.broadcasted_iota(jnp.int32, (k, k), 0)
    c = jax.lax.broadcasted_iota(jnp.int32, (k, k), 1)
    return (r == c).astype(jnp.float32)


def _tc_prep_table(emb_t_pad):
    d, v = emb_t_pad.shape
    nb = v // _TABLE_BLK

    def body(in_ref, out_ref):
        blk_t = jax.lax.dot_general(
            in_ref[...], _eye(d), (((0,), (0,)), ((), ())),
            preferred_element_type=jnp.float32,
        )
        out_ref[...] = jnp.pad(blk_t, ((0, 0), (0, 128 - d)))

    return pl.pallas_call(
        body,
        grid=(nb,),
        in_specs=[pl.BlockSpec((d, _TABLE_BLK), lambda i: (0, i))],
        out_specs=pl.BlockSpec((_TABLE_BLK, 128), lambda i: (i, 0)),
        out_shape=jax.ShapeDtypeStruct((v, 128), jnp.float32),
        compiler_params=pltpu.CompilerParams(
            dimension_semantics=("parallel",),
        ),
    )(emb_t_pad)


def _sc_gather_t(table128, ids2d, s_chunk, d, b):
    n = ids2d.shape[1]
    wpb = b // _GATHER_WIN
    mesh = plsc.VectorSubcoreMesh(core_axis_name="core", subcore_axis_name="subcore")

    @functools.partial(
        pl.kernel,
        out_type=jax.ShapeDtypeStruct((s_chunk, d, b), table128.dtype),
        mesh=mesh,
        scratch_types=[pltpu.VMEM((_GATHER_WIN, 128), jnp.float32)],
        compiler_params=pltpu.CompilerParams(needs_layout_passes=False),
    )
    def gather_kernel(emb_hbm, ids_hbm, out_hbm, g_vmem):
        def body(i_vmem, o_vmem):
            pltpu.sync_copy(emb_hbm.at[i_vmem.at[0]], g_vmem)
            rows = [jax.lax.iota(jnp.int32, 16) + 16 * q for q in range(8)]

            @pl.loop(0, d)
            def _(dd):
                col = jnp.full((16,), dd, jnp.int32)
                for q in range(8):
                    vals = plsc.load_gather(g_vmem, [rows[q], col])
                    o_vmem[0, dd, pl.ds(q * 16, 16)] = vals

        pltpu.emit_pipeline(
            body,
            grid=(n // _GATHER_WIN,),
            in_specs=[pl.BlockSpec((1, _GATHER_WIN), lambda i: (0, i))],
            out_specs=[pl.BlockSpec((1, d, _GATHER_WIN),
                                    lambda i: (i // wpb, 0, i % wpb))],
            core_axis_name=("core", "subcore"),
            dimension_semantics=(pltpu.PARALLEL,),
        )(ids_hbm, out_hbm)

    return gather_kernel(table128, ids2d)


_SB = 4
_NCHUNK = 2


def _tc_combine_chunk(buf, tokens_t, pe3c, topo_t, W, c0, s_chunk):
    s, d, b = tokens_t.shape
    f = topo_t.shape[1]
    cb = c0 // _SB

    def body(*refs):
        if buf is None:
            tok_ref, pe_ref, topo_ref, w_ref, out_ref = refs
        else:
            _, tok_ref, pe_ref, topo_ref, w_ref, out_ref = refs
        for j in range(_SB):
            proj = jax.lax.dot_general(
                w_ref[...], topo_ref[j], (((1,), (0,)), ((), ())),
                preferred_element_type=jnp.float32,
            )
            out_ref[j] = tok_ref[j] + pe_ref[j] + proj

    specs = [
        pl.BlockSpec((_SB, d, b), lambda i: (cb + i, 0, 0)),
        pl.BlockSpec((_SB, d, b), lambda i: (i, 0, 0)),
        pl.BlockSpec((_SB, f, b), lambda i: (cb + i, 0, 0)),
        pl.BlockSpec((d, f), lambda i: (0, 0)),
    ]
    args = (tokens_t, pe3c, topo_t, W)
    aliases = {}
    if buf is not None:
        specs = [pl.BlockSpec(memory_space=pl.ANY)] + specs
        args = (buf,) + args
        aliases = {0: 0}
    return pl.pallas_call(
        body,
        grid=(s_chunk // _SB,),
        in_specs=specs,
        out_specs=pl.BlockSpec((_SB, d, b), lambda i: (cb + i, 0, 0)),
        out_shape=jax.ShapeDtypeStruct((s, d, b), jnp.float32),
        input_output_aliases=aliases,
        compiler_params=pltpu.CompilerParams(
            dimension_semantics=("parallel",),
        ),
    )(*args)


def kernel(tokens, ids, topo_feats, id_emb, W_proj):
    b, s, d = tokens.shape
    n = b * s
    tokens_t = jnp.transpose(tokens, (1, 2, 0))
    topo_t = jnp.transpose(topo_feats, (1, 2, 0))
    ids_sm = ids.T.reshape(1, n).astype(jnp.int32)
    table128 = _tc_prep_table(jnp.pad(id_emb.T, ((0, 0), (0, _VPAD - id_emb.shape[0]))))
    s_chunk = s // _NCHUNK
    nc = b * s_chunk
    pes = [
        _sc_gather_t(table128, ids_sm[:, c * nc:(c + 1) * nc], s_chunk, d, b)
        for c in range(_NCHUNK)
    ]
    buf = None
    for c in range(_NCHUNK):
        buf = _tc_combine_chunk(buf, tokens_t, pes[c], topo_t, W_proj,
                                c * s_chunk, s_chunk)
    return jnp.transpose(buf, (2, 0, 1))


# speedup vs baseline: 1.1491x; 1.1491x over previous
import functools

import jax
import jax.numpy as jnp
from jax.experimental import pallas as pl
from jax.experimental.pallas import tpu as pltpu
from jax.experimental.pallas import tpu_sc as plsc

_GATHER_WIN = 128
_VPAD = 100096
_TABLE_BLK = 4352


def _eye(k):
    r = jax.lax.broadcasted_iota(jnp.int32, (k, k), 0)
    c = jax.lax.broadcasted_iota(jnp.int32, (k, k), 1)
    return (r == c).astype(jnp.float32)


def _tc_prep_table(emb_t_pad):
    d, v = emb_t_pad.shape
    nb = v // _TABLE_BLK

    def body(in_ref, out_ref):
        blk_t = jax.lax.dot_general(
            in_ref[...], _eye(d), (((0,), (0,)), ((), ())),
            preferred_element_type=jnp.float32,
        )
        out_ref[...] = jnp.pad(blk_t, ((0, 0), (0, 128 - d)))

    return pl.pallas_call(
        body,
        grid=(nb,),
        in_specs=[pl.BlockSpec((d, _TABLE_BLK), lambda i: (0, i))],
        out_specs=pl.BlockSpec((_TABLE_BLK, 128), lambda i: (i, 0)),
        out_shape=jax.ShapeDtypeStruct((v, 128), jnp.float32),
        compiler_params=pltpu.CompilerParams(
            dimension_semantics=("parallel",),
        ),
    )(emb_t_pad)


def _sc_gather_t(table128, ids2d, s_chunk, d, b):
    n = ids2d.shape[1]
    nw = n // _GATHER_WIN
    wpw = nw // 32
    ids_rows = ids2d.reshape(32, wpw, _GATHER_WIN)
    mesh = plsc.VectorSubcoreMesh(core_axis_name="core", subcore_axis_name="subcore")

    @functools.partial(
        pl.kernel,
        out_type=jax.ShapeDtypeStruct((s_chunk, d, b), table128.dtype),
        mesh=mesh,
        scratch_types=[
            pltpu.VMEM((wpw, _GATHER_WIN), jnp.int32),
            pltpu.VMEM((2, _GATHER_WIN, 128), jnp.float32),
            pltpu.VMEM((2, d, _GATHER_WIN), jnp.float32),
            pltpu.SemaphoreType.DMA,
            pltpu.SemaphoreType.DMA,
        ],
        compiler_params=pltpu.CompilerParams(needs_layout_passes=False),
    )
    def gather_kernel(emb_hbm, ids_hbm, out_hbm, ids_v, g2, t2, sg, st):
        wid = jax.lax.axis_index("subcore") * 2 + jax.lax.axis_index("core")
        base = wid * wpw
        pltpu.sync_copy(ids_hbm.at[wid], ids_v)
        rows = [jax.lax.iota(jnp.int32, 16) + 16 * q for q in range(8)]

        def out_dest(k):
            g = base + k
            return out_hbm.at[g // 8, :, pl.ds((g % 8) * _GATHER_WIN, _GATHER_WIN)]

        pltpu.async_copy(emb_hbm.at[ids_v.at[0]], g2.at[0], sg)
        for k in range(wpw):
            bb = k % 2
            pltpu.make_async_copy(emb_hbm.at[ids_v.at[k]], g2.at[bb], sg).wait()
            if k + 1 < wpw:
                pltpu.async_copy(emb_hbm.at[ids_v.at[k + 1]], g2.at[1 - bb], sg)
            if k >= 2:
                pltpu.make_async_copy(t2.at[bb], out_dest(k - 2), st).wait()

            gbuf = g2.at[bb]

            @pl.loop(0, d)
            def _(dd):
                col = jnp.full((16,), dd, jnp.int32)
                for q in range(8):
                    vals = plsc.load_gather(gbuf, [rows[q], col])
                    t2[bb, dd, pl.ds(q * 16, 16)] = vals

            pltpu.async_copy(t2.at[bb], out_dest(k), st)
        for k in range(max(wpw - 2, 0), wpw):
            pltpu.make_async_copy(t2.at[k % 2], out_dest(k), st).wait()

    return gather_kernel(table128, ids_rows)


_SB = 4
_NCHUNK = 2


def _tc_combine_chunk(buf, tokens_t, pe3c, topo_t, W, c0, s_chunk):
    s, d, b = tokens_t.shape
    f = topo_t.shape[1]
    cb = c0 // _SB

    def body(*refs):
        if buf is None:
            tok_ref, pe_ref, topo_ref, w_ref, out_ref = refs
        else:
            _, tok_ref, pe_ref, topo_ref, w_ref, out_ref = refs
        for j in range(_SB):
            proj = jax.lax.dot_general(
                w_ref[...], topo_ref[j], (((1,), (0,)), ((), ())),
                preferred_element_type=jnp.float32,
            )
            out_ref[j] = tok_ref[j] + pe_ref[j] + proj

    specs = [
        pl.BlockSpec((_SB, d, b), lambda i: (cb + i, 0, 0)),
        pl.BlockSpec((_SB, d, b), lambda i: (i, 0, 0)),
        pl.BlockSpec((_SB, f, b), lambda i: (cb + i, 0, 0)),
        pl.BlockSpec((d, f), lambda i: (0, 0)),
    ]
    args = (tokens_t, pe3c, topo_t, W)
    aliases = {}
    if buf is not None:
        specs = [pl.BlockSpec(memory_space=pl.ANY)] + specs
        args = (buf,) + args
        aliases = {0: 0}
    return pl.pallas_call(
        body,
        grid=(s_chunk // _SB,),
        in_specs=specs,
        out_specs=pl.BlockSpec((_SB, d, b), lambda i: (cb + i, 0, 0)),
        out_shape=jax.ShapeDtypeStruct((s, d, b), jnp.float32),
        input_output_aliases=aliases,
        compiler_params=pltpu.CompilerParams(
            dimension_semantics=("parallel",),
        ),
    )(*args)


def kernel(tokens, ids, topo_feats, id_emb, W_proj):
    b, s, d = tokens.shape
    n = b * s
    tokens_t = jnp.transpose(tokens, (1, 2, 0))
    topo_t = jnp.transpose(topo_feats, (1, 2, 0))
    ids_sm = ids.T.reshape(1, n).astype(jnp.int32)
    table128 = _tc_prep_table(jnp.pad(id_emb.T, ((0, 0), (0, _VPAD - id_emb.shape[0]))))
    s_chunk = s // _NCHUNK
    nc = b * s_chunk
    pes = [
        _sc_gather_t(table128, ids_sm[:, c * nc:(c + 1) * nc], s_chunk, d, b)
        for c in range(_NCHUNK)
    ]
    buf = None
    for c in range(_NCHUNK):
        buf = _tc_combine_chunk(buf, tokens_t, pes[c], topo_t, W_proj,
                                c * s_chunk, s_chunk)
    return jnp.transpose(buf, (2, 0, 1))


# speedup vs baseline: 2.1278x; 1.8518x over previous
import functools

import jax
import jax.numpy as jnp
from jax.experimental import pallas as pl
from jax.experimental.pallas import tpu as pltpu
from jax.experimental.pallas import tpu_sc as plsc

_GATHER_WIN = 128
_VPAD = 100096
_TABLE_BLK = 4352


def _eye(k):
    r = jax.lax.broadcasted_iota(jnp.int32, (k, k), 0)
    c = jax.lax.broadcasted_iota(jnp.int32, (k, k), 1)
    return (r == c).astype(jnp.float32)


def _tc_prep_table(emb_t_pad):
    d, v = emb_t_pad.shape
    nb = v // _TABLE_BLK

    def body(in_ref, out_ref):
        blk_t = jax.lax.dot_general(
            in_ref[...], _eye(d), (((0,), (0,)), ((), ())),
            preferred_element_type=jnp.float32,
        )
        out_ref[...] = jnp.pad(blk_t, ((0, 0), (0, 128 - d)))

    return pl.pallas_call(
        body,
        grid=(nb,),
        in_specs=[pl.BlockSpec((d, _TABLE_BLK), lambda i: (0, i))],
        out_specs=pl.BlockSpec((_TABLE_BLK, 128), lambda i: (i, 0)),
        out_shape=jax.ShapeDtypeStruct((v, 128), jnp.float32),
        compiler_params=pltpu.CompilerParams(
            dimension_semantics=("parallel",),
        ),
    )(emb_t_pad)


def _sc_gather(table128, ids2d):
    n = ids2d.shape[1]
    dw = table128.shape[1]
    mesh = plsc.VectorSubcoreMesh(core_axis_name="core", subcore_axis_name="subcore")

    @functools.partial(
        pl.kernel,
        out_type=jax.ShapeDtypeStruct((n, dw), table128.dtype),
        mesh=mesh,
    )
    def gather_kernel(emb_hbm, ids_hbm, out_hbm):
        def body(i_vmem, o_vmem):
            pltpu.sync_copy(emb_hbm.at[i_vmem.at[0]], o_vmem)

        pltpu.emit_pipeline(
            body,
            grid=(n // _GATHER_WIN,),
            in_specs=[pl.BlockSpec((1, _GATHER_WIN), lambda i: (0, i))],
            out_specs=[pl.BlockSpec((_GATHER_WIN, dw), lambda i: (i, 0))],
            core_axis_name=("core", "subcore"),
            dimension_semantics=(pltpu.PARALLEL,),
        )(ids_hbm, out_hbm)

    return gather_kernel(table128, ids2d)


_SB = 4
_NCHUNK = 4


def _tc_combine_chunk(buf, tokens_t, pe3c, topo_t, W, c0, s_chunk):
    s, d, b = tokens_t.shape
    f = topo_t.shape[1]
    dw = pe3c.shape[2]
    cb = c0 // _SB

    def body(*refs):
        if buf is None:
            tok_ref, pe_ref, topo_ref, w_ref, out_ref = refs
        else:
            _, tok_ref, pe_ref, topo_ref, w_ref, out_ref = refs
        for j in range(_SB):
            pe_t = jax.lax.dot_general(
                _eye(d), pe_ref[j, :, :d], (((1,), (1,)), ((), ())),
                preferred_element_type=jnp.float32,
                precision=jax.lax.Precision.DEFAULT,
            )
            proj = jax.lax.dot_general(
                w_ref[...], topo_ref[j], (((1,), (0,)), ((), ())),
                preferred_element_type=jnp.float32,
            )
            out_ref[j] = tok_ref[j] + pe_t + proj

    specs = [
        pl.BlockSpec((_SB, d, b), lambda i: (cb + i, 0, 0)),
        pl.BlockSpec((_SB, b, dw), lambda i: (i, 0, 0)),
        pl.BlockSpec((_SB, f, b), lambda i: (cb + i, 0, 0)),
        pl.BlockSpec((d, f), lambda i: (0, 0)),
    ]
    args = (tokens_t, pe3c, topo_t, W)
    aliases = {}
    if buf is not None:
        specs = [pl.BlockSpec(memory_space=pl.ANY)] + specs
        args = (buf,) + args
        aliases = {0: 0}
    return pl.pallas_call(
        body,
        grid=(s_chunk // _SB,),
        in_specs=specs,
        out_specs=pl.BlockSpec((_SB, d, b), lambda i: (cb + i, 0, 0)),
        out_shape=jax.ShapeDtypeStruct((s, d, b), jnp.float32),
        input_output_aliases=aliases,
        compiler_params=pltpu.CompilerParams(
            dimension_semantics=("parallel",),
        ),
    )(*args)


def kernel(tokens, ids, topo_feats, id_emb, W_proj):
    b, s, d = tokens.shape
    n = b * s
    tokens_t = jnp.transpose(tokens, (1, 2, 0))
    topo_t = jnp.transpose(topo_feats, (1, 2, 0))
    ids_sm = ids.T.reshape(1, n).astype(jnp.int32)
    table128 = _tc_prep_table(jnp.pad(id_emb.T, ((0, 0), (0, _VPAD - id_emb.shape[0]))))
    s_chunk = s // _NCHUNK
    nc = b * s_chunk
    pes = [
        _sc_gather(table128, ids_sm[:, c * nc:(c + 1) * nc]).reshape(s_chunk, b, 128)
        for c in range(_NCHUNK)
    ]
    buf = None
    for c in range(_NCHUNK):
        buf = _tc_combine_chunk(buf, tokens_t, pes[c], topo_t, W_proj,
                                c * s_chunk, s_chunk)
    return jnp.transpose(buf, (2, 0, 1))
